# baseline (speedup 1.0000x reference)
from functools import partial

import jax
import jax.numpy as jnp
from jax.experimental import pallas as pl
from jax.experimental.pallas import tpu as pltpu

_EPS = 1e-5
_VMEM_LIMIT = 24 * 1024 * 1024


def _fused_body(xa_ref, xb_ref, gamma_ref, beta_ref, w_ref, b_ref, o_ref, *,
                inv_n, gsize):
    xa = xa_ref[0]
    xb = xb_ref[0]
    C = xa.shape[0] * 2

    s1 = jnp.concatenate(
        [jnp.sum(xa, axis=-1, keepdims=True),
         jnp.sum(xb, axis=-1, keepdims=True)], axis=0)
    s2 = jnp.concatenate(
        [jnp.sum(xa * xa, axis=-1, keepdims=True),
         jnp.sum(xb * xb, axis=-1, keepdims=True)], axis=0)

    row = jax.lax.broadcasted_iota(jnp.int32, (C, C), 0) // gsize
    col = jax.lax.broadcasted_iota(jnp.int32, (C, C), 1) // gsize
    mask = (row == col).astype(jnp.float32)
    s12 = jnp.concatenate([s1, s2], axis=1)
    gs = jnp.dot(mask, s12, preferred_element_type=jnp.float32,
                 precision=jax.lax.Precision.HIGHEST)

    mean = gs[:, 0:1] * inv_n
    ex2 = gs[:, 1:2] * inv_n
    rstd = jax.lax.rsqrt(ex2 - mean * mean + _EPS)
    scale = gamma_ref[...] * rstd
    shift = beta_ref[...] - mean * scale

    h = C // 2
    ya = (xa * scale[:h] + shift[:h]).astype(jnp.bfloat16)
    yb = (xb * scale[h:] + shift[h:]).astype(jnp.bfloat16)
    out = (jnp.dot(w_ref[0], ya, preferred_element_type=jnp.float32) +
           jnp.dot(w_ref[1], yb, preferred_element_type=jnp.float32))
    o_ref[0] = (out + b_ref[...]).astype(o_ref.dtype)


def kernel(x, gamma, beta, w, b):
    B, C, H, W = x.shape
    HW = H * W
    num_groups = C // 4 if C % 4 == 0 else C
    gsize = C // num_groups
    inv_n = 1.0 / float(gsize * HW)
    h = C // 2

    xf = x.reshape(B, C, HW)
    gamma2 = jnp.asarray(gamma, jnp.float32).reshape(C, 1)
    beta2 = jnp.asarray(beta, jnp.float32).reshape(C, 1)
    b2 = jnp.asarray(b, jnp.float32).reshape(C, 1)
    wbf = (jnp.asarray(w).astype(jnp.bfloat16)
           .reshape(C, 2, h).transpose(1, 0, 2))

    out = pl.pallas_call(
        partial(_fused_body, inv_n=inv_n, gsize=gsize),
        out_shape=jax.ShapeDtypeStruct((B, C, HW), jnp.bfloat16),
        grid=(B,),
        in_specs=[
            pl.BlockSpec((1, h, HW), lambda bb: (bb, 0, 0)),
            pl.BlockSpec((1, h, HW), lambda bb: (bb, 1, 0)),
            pl.BlockSpec((C, 1), lambda bb: (0, 0)),
            pl.BlockSpec((C, 1), lambda bb: (0, 0)),
            pl.BlockSpec((2, C, h), lambda bb: (0, 0, 0)),
            pl.BlockSpec((C, 1), lambda bb: (0, 0)),
        ],
        out_specs=pl.BlockSpec((1, C, HW), lambda bb: (bb, 0, 0)),
        compiler_params=pltpu.CompilerParams(
            dimension_semantics=("parallel",),
            vmem_limit_bytes=_VMEM_LIMIT),
    )(xf, xf, gamma2, beta2, wbf, b2)

    return out.reshape(B, C, H, W).astype(x.dtype)


# speedup vs baseline: 1.1301x; 1.1301x over previous
from functools import partial

import jax
import jax.numpy as jnp
from jax.experimental import pallas as pl
from jax.experimental.pallas import tpu as pltpu

_EPS = 1e-5
_VMEM_LIMIT = 24 * 1024 * 1024


def _fused_body(x_ref, mask_ref, gamma_ref, beta_ref, gamma_c_ref, beta_c_ref,
                w_ref, b_ref, o_ref, *, inv_n):
    x = x_ref[0]

    s1 = jnp.sum(x, axis=-1, keepdims=True)
    s2 = jnp.sum(x * x, axis=-1, keepdims=True)
    s12 = jnp.concatenate([s1, s2], axis=1)

    gs_row = jax.lax.dot_general(
        s12, mask_ref[...], (((0,), (0,)), ((), ())),
        preferred_element_type=jnp.float32)
    gs_col = jnp.dot(mask_ref[...], s12,
                     preferred_element_type=jnp.float32)

    mean = gs_row[0:1, :] * inv_n
    ex2 = gs_row[1:2, :] * inv_n
    rstd = jax.lax.rsqrt(ex2 - mean * mean + _EPS)
    scale = gamma_ref[...] * rstd

    mean_c = gs_col[:, 0:1] * inv_n
    ex2_c = gs_col[:, 1:2] * inv_n
    rstd_c = jax.lax.rsqrt(ex2_c - mean_c * mean_c + _EPS)
    scale_c = gamma_c_ref[...] * rstd_c
    shift_c = beta_c_ref[...] - mean_c * scale_c

    wf = w_ref[...]
    wp = (wf * scale).astype(jnp.bfloat16)
    biasp = jnp.dot(wf, shift_c,
                    preferred_element_type=jnp.float32) + b_ref[...]

    out = jnp.dot(wp, x.astype(jnp.bfloat16),
                  preferred_element_type=jnp.float32)
    o_ref[0] = (out + biasp).astype(o_ref.dtype)


def kernel(x, gamma, beta, w, b):
    B, C, H, W = x.shape
    HW = H * W
    num_groups = C // 4 if C % 4 == 0 else C
    gsize = C // num_groups
    inv_n = 1.0 / float(gsize * HW)

    xf = x.reshape(B, C, HW)
    cid = jnp.arange(C, dtype=jnp.int32) // gsize
    mask = (cid[:, None] == cid[None, :]).astype(jnp.float32)
    gamma2 = jnp.asarray(gamma, jnp.float32).reshape(1, C)
    beta2 = jnp.asarray(beta, jnp.float32).reshape(1, C)
    gamma_c = jnp.asarray(gamma, jnp.float32).reshape(C, 1)
    beta_c = jnp.asarray(beta, jnp.float32).reshape(C, 1)
    b2 = jnp.asarray(b, jnp.float32).reshape(C, 1)
    wf = jnp.asarray(w, jnp.float32)

    out = pl.pallas_call(
        partial(_fused_body, inv_n=inv_n),
        out_shape=jax.ShapeDtypeStruct((B, C, HW), jnp.bfloat16),
        grid=(B,),
        in_specs=[
            pl.BlockSpec((1, C, HW), lambda bb: (bb, 0, 0)),
            pl.BlockSpec((C, C), lambda bb: (0, 0)),
            pl.BlockSpec((1, C), lambda bb: (0, 0)),
            pl.BlockSpec((1, C), lambda bb: (0, 0)),
            pl.BlockSpec((C, 1), lambda bb: (0, 0)),
            pl.BlockSpec((C, 1), lambda bb: (0, 0)),
            pl.BlockSpec((C, C), lambda bb: (0, 0)),
            pl.BlockSpec((C, 1), lambda bb: (0, 0)),
        ],
        out_specs=pl.BlockSpec((1, C, HW), lambda bb: (bb, 0, 0)),
        compiler_params=pltpu.CompilerParams(
            dimension_semantics=("arbitrary",),
            vmem_limit_bytes=_VMEM_LIMIT),
    )(xf, mask, gamma2, beta2, gamma_c, beta_c, wf, b2)

    return out.reshape(B, C, H, W).astype(x.dtype)


# speedup vs baseline: 1.1306x; 1.0004x over previous
from functools import partial

import jax
import jax.numpy as jnp
from jax.experimental import pallas as pl
from jax.experimental.pallas import tpu as pltpu

_EPS = 1e-5
_VMEM_LIMIT = 24 * 1024 * 1024


def _fused_body(x_ref, mask_ref, gamma_ref, beta_ref, gamma_c_ref, beta_c_ref,
                w_ref, b_ref, o_ref, *, inv_n):
    x = x_ref[0]

    s1 = jnp.sum(x, axis=-1, keepdims=True)
    s2 = jnp.sum(x * x, axis=-1, keepdims=True)
    s12 = jnp.concatenate([s1, s2], axis=1)

    gs_row = jax.lax.dot_general(
        s12, mask_ref[...], (((0,), (0,)), ((), ())),
        preferred_element_type=jnp.float32)
    gs_col = jnp.dot(mask_ref[...], s12,
                     preferred_element_type=jnp.float32)

    mean = gs_row[0:1, :] * inv_n
    ex2 = gs_row[1:2, :] * inv_n
    rstd = jax.lax.rsqrt(ex2 - mean * mean + _EPS)
    scale = gamma_ref[...] * rstd

    mean_c = gs_col[:, 0:1] * inv_n
    ex2_c = gs_col[:, 1:2] * inv_n
    rstd_c = jax.lax.rsqrt(ex2_c - mean_c * mean_c + _EPS)
    scale_c = gamma_c_ref[...] * rstd_c
    shift_c = beta_c_ref[...] - mean_c * scale_c

    wf = w_ref[...]
    wp = (wf * scale).astype(jnp.bfloat16)
    biasp = jnp.dot(wf, shift_c,
                    preferred_element_type=jnp.float32) + b_ref[...]

    out = jnp.dot(wp, x.astype(jnp.bfloat16),
                  preferred_element_type=jnp.float32)
    o_ref[0] = (out + biasp).astype(o_ref.dtype)


def kernel(x, gamma, beta, w, b):
    B, C, H, W = x.shape
    HW = H * W
    num_groups = C // 4 if C % 4 == 0 else C
    gsize = C // num_groups
    inv_n = 1.0 / float(gsize * HW)

    xf = x.reshape(B, C, HW)
    cid = jnp.arange(C, dtype=jnp.int32) // gsize
    mask = (cid[:, None] == cid[None, :]).astype(jnp.float32)
    gamma2 = jnp.asarray(gamma, jnp.float32).reshape(1, C)
    beta2 = jnp.asarray(beta, jnp.float32).reshape(1, C)
    gamma_c = jnp.asarray(gamma, jnp.float32).reshape(C, 1)
    beta_c = jnp.asarray(beta, jnp.float32).reshape(C, 1)
    b2 = jnp.asarray(b, jnp.float32).reshape(C, 1)
    wf = jnp.asarray(w, jnp.float32)

    out = pl.pallas_call(
        partial(_fused_body, inv_n=inv_n),
        out_shape=jax.ShapeDtypeStruct((B, C, HW), jnp.bfloat16),
        grid=(B,),
        in_specs=[
            pl.BlockSpec((1, C, HW), lambda bb: (bb, 0, 0)),
            pl.BlockSpec((C, C), lambda bb: (0, 0)),
            pl.BlockSpec((1, C), lambda bb: (0, 0)),
            pl.BlockSpec((1, C), lambda bb: (0, 0)),
            pl.BlockSpec((C, 1), lambda bb: (0, 0)),
            pl.BlockSpec((C, 1), lambda bb: (0, 0)),
            pl.BlockSpec((C, C), lambda bb: (0, 0)),
            pl.BlockSpec((C, 1), lambda bb: (0, 0)),
        ],
        out_specs=pl.BlockSpec((1, C, HW), lambda bb: (bb, 0, 0)),
        compiler_params=pltpu.CompilerParams(
            dimension_semantics=("arbitrary",),
            allow_input_fusion=[True] + [False] * 7,
            vmem_limit_bytes=_VMEM_LIMIT),
    )(xf, mask, gamma2, beta2, gamma_c, beta_c, wf, b2)

    return out.reshape(B, C, H, W).astype(x.dtype)
